# baseline (speedup 1.0000x reference)
import functools

import jax
import jax.numpy as jnp
from jax import lax
from jax.experimental import pallas as pl
from jax.experimental.pallas import tpu as pltpu
from jax.experimental.pallas import tpu_sc as plsc

N_ATOMS = 100000
D_FEAT = 128
N_SPECIES = 100

NUM_CORES = 2
NUM_SUBCORES = 16
LANES = 16

SC_PER_W = 4000
SC_ACTIVE_W = N_ATOMS // SC_PER_W

N_GROUPS = N_ATOMS // 8
GRP_BLOCK = 1250
NUM_BLOCKS = N_GROUPS // GRP_BLOCK


def _sc_gather_scales(z32, scales):
    mesh = plsc.VectorSubcoreMesh(
        core_axis_name="c",
        subcore_axis_name="s",
        num_cores=NUM_CORES,
        num_subcores=NUM_SUBCORES,
    )

    @functools.partial(
        pl.kernel,
        out_type=jax.ShapeDtypeStruct((N_ATOMS,), jnp.float32),
        mesh=mesh,
        compiler_params=pltpu.CompilerParams(needs_layout_passes=False),
        scratch_types=[
            pltpu.VMEM((SC_PER_W,), jnp.int32),
            pltpu.VMEM((SC_PER_W,), jnp.float32),
            pltpu.VMEM((N_SPECIES,), jnp.float32),
        ],
    )
    def gather_kernel(z_hbm, scales_hbm, out_hbm, idx_v, s_v, tab_v):
        wid = lax.axis_index("s") * NUM_CORES + lax.axis_index("c")

        @pl.when(wid < SC_ACTIVE_W)
        def _():
            base = wid * SC_PER_W
            pltpu.sync_copy(scales_hbm, tab_v)
            pltpu.sync_copy(z_hbm.at[pl.ds(base, SC_PER_W)], idx_v)

            def body(i, carry):
                idx = idx_v[pl.ds(i * LANES, LANES)]
                s_v[pl.ds(i * LANES, LANES)] = plsc.load_gather(tab_v, [idx])
                return carry

            lax.fori_loop(0, SC_PER_W // LANES, body, 0, unroll=4)
            pltpu.sync_copy(s_v, out_hbm.at[pl.ds(base, SC_PER_W)])

    return gather_kernel(z32, scales)


def _tc_mul_kernel(x_ref, s_ref, out_ref):
    i = pl.program_id(0)
    s_blk = s_ref[pl.ds(i * GRP_BLOCK, GRP_BLOCK), :]
    out_ref[...] = x_ref[...] * s_blk[:, :, None]


def _tc_scale(x3, s2):
    return pl.pallas_call(
        _tc_mul_kernel,
        grid=(NUM_BLOCKS,),
        in_specs=[
            pl.BlockSpec((GRP_BLOCK, 8, D_FEAT), lambda i: (i, 0, 0)),
            pl.BlockSpec((N_GROUPS, 8), lambda i: (0, 0)),
        ],
        out_specs=pl.BlockSpec((GRP_BLOCK, 8, D_FEAT), lambda i: (i, 0, 0)),
        out_shape=jax.ShapeDtypeStruct((N_GROUPS, 8, D_FEAT), jnp.float32),
    )(x3, s2)


def kernel(x, Z, scales):
    z32 = Z.astype(jnp.int32)
    s = _sc_gather_scales(z32, scales)
    x3 = x.reshape(N_GROUPS, 8, D_FEAT)
    s2 = s.reshape(N_GROUPS, 8)
    out3 = _tc_scale(x3, s2)
    return out3.reshape(N_ATOMS, D_FEAT)


# speedup vs baseline: 1.0013x; 1.0013x over previous
import functools

import jax
import jax.numpy as jnp
from jax import lax
from jax.experimental import pallas as pl
from jax.experimental.pallas import tpu as pltpu
from jax.experimental.pallas import tpu_sc as plsc

N_ATOMS = 100000
D_FEAT = 128
N_SPECIES = 100

NUM_CORES = 2
NUM_SUBCORES = 16
LANES = 16

SC_PER_W = 4000
SC_ACTIVE_W = N_ATOMS // SC_PER_W

N_GROUPS = N_ATOMS // 8
GRP_BLOCK = 2500
NUM_BLOCKS = N_GROUPS // GRP_BLOCK


def _sc_gather_scales(z32, scales):
    mesh = plsc.VectorSubcoreMesh(
        core_axis_name="c",
        subcore_axis_name="s",
        num_cores=NUM_CORES,
        num_subcores=NUM_SUBCORES,
    )

    @functools.partial(
        pl.kernel,
        out_type=jax.ShapeDtypeStruct((N_ATOMS,), jnp.float32),
        mesh=mesh,
        compiler_params=pltpu.CompilerParams(needs_layout_passes=False),
        scratch_types=[
            pltpu.VMEM((SC_PER_W,), jnp.int32),
            pltpu.VMEM((SC_PER_W,), jnp.float32),
            pltpu.VMEM((N_SPECIES,), jnp.float32),
        ],
    )
    def gather_kernel(z_hbm, scales_hbm, out_hbm, idx_v, s_v, tab_v):
        wid = lax.axis_index("s") * NUM_CORES + lax.axis_index("c")

        @pl.when(wid < SC_ACTIVE_W)
        def _():
            base = wid * SC_PER_W
            pltpu.sync_copy(scales_hbm, tab_v)
            pltpu.sync_copy(z_hbm.at[pl.ds(base, SC_PER_W)], idx_v)

            def body(i, carry):
                idx = idx_v[pl.ds(i * LANES, LANES)]
                s_v[pl.ds(i * LANES, LANES)] = plsc.load_gather(tab_v, [idx])
                return carry

            lax.fori_loop(0, SC_PER_W // LANES, body, 0, unroll=4)
            pltpu.sync_copy(s_v, out_hbm.at[pl.ds(base, SC_PER_W)])

    return gather_kernel(z32, scales)


def _tc_mul_kernel(x_ref, s_ref, out_ref):
    i = pl.program_id(0)
    s_blk = s_ref[pl.ds(i * GRP_BLOCK, GRP_BLOCK), :]
    out_ref[...] = x_ref[...] * s_blk[:, :, None]


def _tc_scale(x3, s2):
    return pl.pallas_call(
        _tc_mul_kernel,
        grid=(NUM_BLOCKS,),
        in_specs=[
            pl.BlockSpec((GRP_BLOCK, 8, D_FEAT), lambda i: (i, 0, 0)),
            pl.BlockSpec((N_GROUPS, 8), lambda i: (0, 0)),
        ],
        out_specs=pl.BlockSpec((GRP_BLOCK, 8, D_FEAT), lambda i: (i, 0, 0)),
        out_shape=jax.ShapeDtypeStruct((N_GROUPS, 8, D_FEAT), jnp.float32),
    )(x3, s2)


def kernel(x, Z, scales):
    z32 = Z.astype(jnp.int32)
    s = _sc_gather_scales(z32, scales)
    x3 = x.reshape(N_GROUPS, 8, D_FEAT)
    s2 = s.reshape(N_GROUPS, 8)
    out3 = _tc_scale(x3, s2)
    return out3.reshape(N_ATOMS, D_FEAT)
